# baseline (speedup 1.0000x reference)
import functools

import jax
import jax.numpy as jnp
from jax import lax
from jax.experimental import pallas as pl
from jax.experimental.pallas import tpu as pltpu
from jax.experimental.pallas import tpu_sc as plsc

_DIM = 64
_ROWS = 40960
_NEG = 5
_BATCH = 4096

_NC = 2
_NS = 16
_NW = _NC * _NS
_G = 16
_RPW = _ROWS // _NW
_CH = _RPW // _G
_NPW = _RPW * _NEG
_BPW = _RPW // 128
_NBPW = _NPW // 128


def _sc_scores(pu, pv, nv, u2, v2):
  mesh = plsc.VectorSubcoreMesh(core_axis_name="c", subcore_axis_name="s")

  @functools.partial(
      pl.kernel,
      out_type=[
          jax.ShapeDtypeStruct((_ROWS,), jnp.float32),
          jax.ShapeDtypeStruct((_ROWS,), jnp.float32),
      ],
      mesh=mesh,
      compiler_params=pltpu.CompilerParams(needs_layout_passes=False),
      scratch_types=[
          pltpu.VMEM((2 * _RPW,), jnp.int32),
          pltpu.VMEM((2 * _RPW,), jnp.int32),
          pltpu.VMEM((2 * _NPW,), jnp.int32),
          pltpu.VMEM((_RPW,), jnp.int32),
          pltpu.VMEM((_RPW,), jnp.int32),
          pltpu.VMEM((_RPW,), jnp.int32),
          pltpu.VMEM((_RPW,), jnp.int32),
          pltpu.VMEM((_NPW,), jnp.int32),
          pltpu.VMEM((_NPW,), jnp.int32),
          pltpu.VMEM((_RPW,), jnp.int32),
          pltpu.VMEM((_RPW,), jnp.int32),
          pltpu.VMEM((_RPW,), jnp.int32),
          pltpu.VMEM((_RPW,), jnp.int32),
          pltpu.VMEM((_NPW,), jnp.int32),
          pltpu.VMEM((_NPW,), jnp.int32),
          pltpu.VMEM((2, _G, 128), jnp.float32),
          pltpu.VMEM((2, _G, 128), jnp.float32),
          pltpu.VMEM((2, _G, 128), jnp.float32),
          pltpu.VMEM((2, _G, 128), jnp.float32),
          pltpu.VMEM((2, _NEG * _G, 128), jnp.float32),
          pltpu.VMEM((2, _NEG * _G, 128), jnp.float32),
          pltpu.VMEM((16, 6 * _G), jnp.float32),
          pltpu.VMEM((_RPW,), jnp.float32),
          pltpu.VMEM((_RPW,), jnp.float32),
          pltpu.SemaphoreType.DMA((2,)),
          pltpu.SemaphoreType.DMA((2,)),
          pltpu.SemaphoreType.DMA((2,)),
          pltpu.SemaphoreType.DMA((2,)),
          pltpu.SemaphoreType.DMA((2,)),
          pltpu.SemaphoreType.DMA((2,)),
      ],
  )
  def k(pu_hbm, pv_hbm, nv_hbm, u_hbm, v_hbm, score_hbm, negsum_hbm,
        raw_u, raw_v, raw_n,
        hu0, hu1, hv0, hv1, hn0, hn1,
        pu0, pu1, pv0, pv1, pn0, pn1,
        gu0, gu1, gv0, gv1, gn0, gn1,
        partt, score_all, negsum,
        su0, su1, sv0, sv1, sn0, sn1):
    wid = lax.axis_index("s") * _NC + lax.axis_index("c")
    lanes = lax.iota(jnp.int32, 16)

    pltpu.sync_copy(pu_hbm.at[pl.ds(wid * 2 * _RPW, 2 * _RPW)], raw_u)
    pltpu.sync_copy(pv_hbm.at[pl.ds(wid * 2 * _RPW, 2 * _RPW)], raw_v)
    pltpu.sync_copy(nv_hbm.at[pl.ds(wid * 2 * _NPW, 2 * _NPW)], raw_n)

    for b in range(_BPW):
      for t in range(8):
        d = pl.ds(b * 128 + 16 * t, 16)
        s0 = raw_u[pl.ds(b * 256 + 16 * t, 16)]
        hu0[d] = lax.shift_right_logical(s0, 1)
        pu0[d] = (s0 & 1) * 64
        s1 = raw_u[pl.ds(b * 256 + 128 + 16 * t, 16)]
        hu1[d] = lax.shift_right_logical(s1, 1)
        pu1[d] = (s1 & 1) * 64
        s2 = raw_v[pl.ds(b * 256 + 16 * t, 16)]
        hv0[d] = lax.shift_right_logical(s2, 1)
        pv0[d] = (s2 & 1) * 64
        s3 = raw_v[pl.ds(b * 256 + 128 + 16 * t, 16)]
        hv1[d] = lax.shift_right_logical(s3, 1)
        pv1[d] = (s3 & 1) * 64
    for b in range(_NBPW):
      for t in range(8):
        d = pl.ds(b * 128 + 16 * t, 16)
        s0 = raw_n[pl.ds(b * 256 + 16 * t, 16)]
        hn0[d] = lax.shift_right_logical(s0, 1)
        pn0[d] = (s0 & 1) * 64
        s1 = raw_n[pl.ds(b * 256 + 128 + 16 * t, 16)]
        hn1[d] = lax.shift_right_logical(s1, 1)
        pn1[d] = (s1 & 1) * 64

    def copies(c, s):
      du = pl.ds(c * _G, _G)
      dn = pl.ds(c * _NEG * _G, _NEG * _G)
      return [
          pltpu.make_async_copy(u_hbm.at[hu0.at[du]], gu0.at[s], su0.at[s]),
          pltpu.make_async_copy(u_hbm.at[hu1.at[du]], gu1.at[s], su1.at[s]),
          pltpu.make_async_copy(v_hbm.at[hv0.at[du]], gv0.at[s], sv0.at[s]),
          pltpu.make_async_copy(v_hbm.at[hv1.at[du]], gv1.at[s], sv1.at[s]),
          pltpu.make_async_copy(v_hbm.at[hn0.at[dn]], gn0.at[s], sn0.at[s]),
          pltpu.make_async_copy(v_hbm.at[hn1.at[dn]], gn1.at[s], sn1.at[s]),
      ]

    for cp in copies(0, 0):
      cp.start()

    def chunk(c, carry):
      s = lax.rem(c, 2)

      @pl.when(c + 1 < _CH)
      def _():
        for cp in copies(c + 1, 1 - s):
          cp.start()

      for cp in copies(c, s):
        cp.wait()

      du = pl.ds(c * _G, _G)
      ou0 = pu0[du]
      ou1 = pu1[du]
      ov0 = pv0[du]
      ov1 = pv1[du]
      on0 = [pn0[pl.ds(c * _NEG * _G + 16 * t, 16)] for t in range(_NEG)]
      on1 = [pn1[pl.ds(c * _NEG * _G + 16 * t, 16)] for t in range(_NEG)]

      for i in range(_G):
        a0 = ou0[i]
        a1 = ou1[i]
        b0 = ov0[i]
        b1 = ov1[i]
        su = [gu0[s, i, pl.ds(a0 + 16 * t, 16)]
              + gu1[s, i, pl.ds(a1 + 16 * t, 16)]
              for t in range(4)]
        p = su[0] * (gv0[s, i, pl.ds(b0, 16)] + gv1[s, i, pl.ds(b1, 16)])
        for t in range(1, 4):
          p = p + su[t] * (gv0[s, i, pl.ds(b0 + 16 * t, 16)]
                           + gv1[s, i, pl.ds(b1 + 16 * t, 16)])
        plsc.store_scatter(partt, [lanes, jnp.full((16,), i, jnp.int32)], p)
        for kk in range(_NEG):
          r = _NEG * i + kk
          c0 = on0[r // 16][r % 16]
          c1 = on1[r // 16][r % 16]
          pn = su[0] * (gn0[s, r, pl.ds(c0, 16)]
                        + gn1[s, r, pl.ds(c1, 16)])
          for t in range(1, 4):
            pn = pn + su[t] * (gn0[s, r, pl.ds(c0 + 16 * t, 16)]
                               + gn1[s, r, pl.ds(c1 + 16 * t, 16)])
          plsc.store_scatter(
              partt, [lanes, jnp.full((16,), (1 + kk) * _G + i, jnp.int32)],
              pn)

      acc = []
      for q in range(1 + _NEG):
        a = partt[0, pl.ds(q * _G, _G)]
        for t in range(1, 16):
          a = a + partt[t, pl.ds(q * _G, _G)]
        acc.append(a)
      ds = pl.ds(c * _G, _G)
      score_all[ds] = acc[0] * 0.25
      e = jnp.exp(acc[1] * 0.25)
      for kk in range(2, 1 + _NEG):
        e = e + jnp.exp(acc[kk] * 0.25)
      negsum[ds] = e
      return carry

    lax.fori_loop(0, _CH, chunk, 0)

    pltpu.sync_copy(score_all, score_hbm.at[pl.ds(wid * _RPW, _RPW)])
    pltpu.sync_copy(negsum, negsum_hbm.at[pl.ds(wid * _RPW, _RPW)])

  return k(pu, pv, nv, u2, v2)


def _tc_loss(score2d, negsum2d):
  def body(s_ref, n_ref, o_ref):
    val = (jnp.sum(jnp.log(1.0 + n_ref[...]))
           - jnp.sum(s_ref[...])) * (1.0 / _BATCH)
    o_ref[...] = jnp.broadcast_to(val, (1, 1))

  return pl.pallas_call(
      body,
      out_shape=jax.ShapeDtypeStruct((1, 1), jnp.float32),
  )(score2d, negsum2d)


def _blockview(idx):
  n = idx.shape[0]
  return jnp.swapaxes(idx.reshape(n // 128, 128, 2), 1, 2).reshape(-1)


def kernel(pos_u, pos_v, neg_v, u_weight, v_weight):
  pu = _blockview(pos_u)
  pv = _blockview(pos_v)
  nv = _blockview(neg_v)
  u2 = u_weight.reshape(-1, 128)
  v2 = v_weight.reshape(-1, 128)
  score, negsum = _sc_scores(pu, pv, nv, u2, v2)
  loss = _tc_loss(score.reshape(_ROWS // 128, 128),
                  negsum.reshape(_ROWS // 128, 128))
  return loss[0, 0]


# speedup vs baseline: 1.0813x; 1.0813x over previous
import functools

import jax
import jax.numpy as jnp
from jax import lax
from jax.experimental import pallas as pl
from jax.experimental.pallas import tpu as pltpu
from jax.experimental.pallas import tpu_sc as plsc

_DIM = 64
_ROWS = 40960
_NEG = 5
_BATCH = 4096

_NC = 2
_NS = 16
_NW = _NC * _NS
_G = 16
_RPW = _ROWS // _NW
_CH = _RPW // _G
_NPW = _RPW * _NEG
_BPW = _RPW // 128
_NBPW = _NPW // 128


def _sc_scores(pu, pv, nv, u_weight, v_weight):
  mesh = plsc.VectorSubcoreMesh(core_axis_name="c", subcore_axis_name="s")

  @functools.partial(
      pl.kernel,
      out_type=[
          jax.ShapeDtypeStruct((_ROWS,), jnp.float32),
          jax.ShapeDtypeStruct((_ROWS,), jnp.float32),
      ],
      mesh=mesh,
      compiler_params=pltpu.CompilerParams(needs_layout_passes=False),
      scratch_types=[
          pltpu.VMEM((2 * _RPW,), jnp.int32),
          pltpu.VMEM((2 * _RPW,), jnp.int32),
          pltpu.VMEM((2 * _NPW,), jnp.int32),
          pltpu.VMEM((_RPW,), jnp.int32),
          pltpu.VMEM((_RPW,), jnp.int32),
          pltpu.VMEM((_RPW,), jnp.int32),
          pltpu.VMEM((_RPW,), jnp.int32),
          pltpu.VMEM((_NPW,), jnp.int32),
          pltpu.VMEM((_NPW,), jnp.int32),
          pltpu.VMEM((2, _G, 128), jnp.float32),
          pltpu.VMEM((2, _G, 128), jnp.float32),
          pltpu.VMEM((2, _G, 128), jnp.float32),
          pltpu.VMEM((2, _G, 128), jnp.float32),
          pltpu.VMEM((2, _NEG * _G, 128), jnp.float32),
          pltpu.VMEM((2, _NEG * _G, 128), jnp.float32),
          pltpu.VMEM((16, 6 * _G), jnp.float32),
          pltpu.VMEM((_RPW,), jnp.float32),
          pltpu.VMEM((_RPW,), jnp.float32),
          pltpu.SemaphoreType.DMA((2,)),
          pltpu.SemaphoreType.DMA((2,)),
          pltpu.SemaphoreType.DMA((2,)),
          pltpu.SemaphoreType.DMA((2,)),
          pltpu.SemaphoreType.DMA((2,)),
          pltpu.SemaphoreType.DMA((2,)),
      ],
  )
  def k(pu_hbm, pv_hbm, nv_hbm, u_hbm, v_hbm, score_hbm, negsum_hbm,
        raw_u, raw_v, raw_n,
        iu0, iu1, iv0, iv1, in0, in1,
        gu0, gu1, gv0, gv1, gn0, gn1,
        partt, score_all, negsum,
        su0, su1, sv0, sv1, sn0, sn1):
    wid = lax.axis_index("s") * _NC + lax.axis_index("c")
    lanes = lax.iota(jnp.int32, 16)

    pltpu.sync_copy(pu_hbm.at[pl.ds(wid * 2 * _RPW, 2 * _RPW)], raw_u)
    pltpu.sync_copy(pv_hbm.at[pl.ds(wid * 2 * _RPW, 2 * _RPW)], raw_v)
    pltpu.sync_copy(nv_hbm.at[pl.ds(wid * 2 * _NPW, 2 * _NPW)], raw_n)

    for b in range(_BPW):
      for t in range(8):
        d = pl.ds(b * 128 + 16 * t, 16)
        iu0[d] = raw_u[pl.ds(b * 256 + 16 * t, 16)]
        iu1[d] = raw_u[pl.ds(b * 256 + 128 + 16 * t, 16)]
        iv0[d] = raw_v[pl.ds(b * 256 + 16 * t, 16)]
        iv1[d] = raw_v[pl.ds(b * 256 + 128 + 16 * t, 16)]
    for b in range(_NBPW):
      for t in range(8):
        d = pl.ds(b * 128 + 16 * t, 16)
        in0[d] = raw_n[pl.ds(b * 256 + 16 * t, 16)]
        in1[d] = raw_n[pl.ds(b * 256 + 128 + 16 * t, 16)]

    def copies(c, s):
      du = pl.ds(c * _G, _G)
      dn = pl.ds(c * _NEG * _G, _NEG * _G)
      return [
          pltpu.make_async_copy(u_hbm.at[iu0.at[du]], gu0.at[s], su0.at[s]),
          pltpu.make_async_copy(u_hbm.at[iu1.at[du]], gu1.at[s], su1.at[s]),
          pltpu.make_async_copy(v_hbm.at[iv0.at[du]], gv0.at[s], sv0.at[s]),
          pltpu.make_async_copy(v_hbm.at[iv1.at[du]], gv1.at[s], sv1.at[s]),
          pltpu.make_async_copy(v_hbm.at[in0.at[dn]], gn0.at[s], sn0.at[s]),
          pltpu.make_async_copy(v_hbm.at[in1.at[dn]], gn1.at[s], sn1.at[s]),
      ]

    for cp in copies(0, 0):
      cp.start()

    def chunk(c, carry):
      s = lax.rem(c, 2)

      @pl.when(c + 1 < _CH)
      def _():
        for cp in copies(c + 1, 1 - s):
          cp.start()

      for cp in copies(c, s):
        cp.wait()

      for i in range(_G):
        su = [gu0[s, i, pl.ds(16 * t, 16)] + gu1[s, i, pl.ds(16 * t, 16)]
              for t in range(4)]
        p = su[0] * (gv0[s, i, pl.ds(0, 16)] + gv1[s, i, pl.ds(0, 16)])
        for t in range(1, 4):
          p = p + su[t] * (gv0[s, i, pl.ds(16 * t, 16)]
                           + gv1[s, i, pl.ds(16 * t, 16)])
        plsc.store_scatter(partt, [lanes, jnp.full((16,), i, jnp.int32)], p)
        for kk in range(_NEG):
          r = _NEG * i + kk
          pn = su[0] * (gn0[s, r, pl.ds(0, 16)] + gn1[s, r, pl.ds(0, 16)])
          for t in range(1, 4):
            pn = pn + su[t] * (gn0[s, r, pl.ds(16 * t, 16)]
                               + gn1[s, r, pl.ds(16 * t, 16)])
          plsc.store_scatter(
              partt, [lanes, jnp.full((16,), (1 + kk) * _G + i, jnp.int32)],
              pn)

      acc = []
      for q in range(1 + _NEG):
        a = partt[0, pl.ds(q * _G, _G)]
        for t in range(1, 16):
          a = a + partt[t, pl.ds(q * _G, _G)]
        acc.append(a)
      ds = pl.ds(c * _G, _G)
      score_all[ds] = acc[0] * 0.25
      e = jnp.exp(acc[1] * 0.25)
      for kk in range(2, 1 + _NEG):
        e = e + jnp.exp(acc[kk] * 0.25)
      negsum[ds] = e
      return carry

    lax.fori_loop(0, _CH, chunk, 0)

    pltpu.sync_copy(score_all, score_hbm.at[pl.ds(wid * _RPW, _RPW)])
    pltpu.sync_copy(negsum, negsum_hbm.at[pl.ds(wid * _RPW, _RPW)])

  return k(pu, pv, nv, u_weight, v_weight)


def _tc_loss(score2d, negsum2d):
  def body(s_ref, n_ref, o_ref):
    val = (jnp.sum(jnp.log(1.0 + n_ref[...]))
           - jnp.sum(s_ref[...])) * (1.0 / _BATCH)
    o_ref[...] = jnp.broadcast_to(val, (1, 1))

  return pl.pallas_call(
      body,
      out_shape=jax.ShapeDtypeStruct((1, 1), jnp.float32),
  )(score2d, negsum2d)


def _blockview(idx):
  n = idx.shape[0]
  return jnp.swapaxes(idx.reshape(n // 128, 128, 2), 1, 2).reshape(-1)


def kernel(pos_u, pos_v, neg_v, u_weight, v_weight):
  pu = _blockview(pos_u)
  pv = _blockview(pos_v)
  nv = _blockview(neg_v)
  u2 = jnp.pad(u_weight, ((0, 0), (0, 64)))
  v2 = jnp.pad(v_weight, ((0, 0), (0, 64)))
  score, negsum = _sc_scores(pu, pv, nv, u2, v2)
  loss = _tc_loss(score.reshape(_ROWS // 128, 128),
                  negsum.reshape(_ROWS // 128, 128))
  return loss[0, 0]
